# Initial kernel scaffold; baseline (speedup 1.0000x reference)
import jax
import jax.numpy as jnp
from jax.experimental import pallas as pl


def kernel(x, c1_wc, c1_wd, c1_s, c1_b, c2_wc, c2_wd, c2_s, c2_b, c3_wc, c3_wd, c3_s, c3_b, c4_wc, c4_wd, c4_s, c4_b, c5_a, c5_b2, c5_c, c5_d, c5_s, c5_bias, w1g, sg1_a, sg1_b, sg1_c, sg1_d, sg1_s, sg1_bias, sg2_w, sg2_s, sg2_bias, w3p):
    raise NotImplementedError("write your pallas kernel here")


# speedup vs baseline: 1.1143x; 1.1143x over previous
import functools

import jax
import jax.numpy as jnp
from jax import lax
from jax.experimental import pallas as pl
from jax.experimental.pallas import tpu as pltpu

LEAKY_SLOPE = 0.2

_VMEM_LIMIT_BYTES = int(min((64 * 1024 * 1024) * 3 // 4, 100 * 1024 * 1024))


def _cparams(semantics):
    return pltpu.CompilerParams(dimension_semantics=semantics,
                                vmem_limit_bytes=_VMEM_LIMIT_BYTES)


def _lrelu(y):
    return jnp.where(y > 0, y, LEAKY_SLOPE * y)


def _edge_conv_body(x_ref, wc_ref, wd_ref, s_ref, b_ref, out_ref, *, k):
    n = x_ref.shape[0]
    tn, cout = out_ref.shape
    row0 = pl.multiple_of(pl.program_id(1) * tn, tn)

    x_all = x_ref[...]
    xf_all = x_all.astype(jnp.float32)
    xb_all = x_all.astype(jnp.bfloat16)
    x_tile = x_ref[pl.ds(row0, tn), :]
    xt_f = x_tile.astype(jnp.float32)
    xt_b = x_tile.astype(jnp.bfloat16)

    inner = lax.dot_general(xt_f, xf_all, (((1,), (1,)), ((), ())),
                            preferred_element_type=jnp.float32)
    sq_t = jnp.sum(xt_f * xt_f, axis=1, keepdims=True)
    sq_a = jnp.sum(xf_all * xf_all, axis=1, keepdims=True)
    d = 2.0 * inner - sq_t - sq_a.T

    lane = lax.broadcasted_iota(jnp.int32, (tn, n), 1)

    wd = wd_ref[...]
    self_lane = row0 + lax.broadcasted_iota(jnp.int32, (tn, 1), 0)
    d = jnp.where(lane == self_lane, -jnp.inf, d)

    base = (jnp.dot(xt_b, wc_ref[...], preferred_element_type=jnp.float32)
            - jnp.dot(xt_b, wd, preferred_element_type=jnp.float32))

    w_self = jnp.dot(xt_b, wd, preferred_element_type=jnp.float32)
    wmax = w_self
    wmin = w_self

    for _ in range(k - 1):
        idx = jnp.argmax(d, axis=1, keepdims=True)
        hit = lane == idx
        d = jnp.where(hit, -jnp.inf, d)
        onehot = jnp.where(hit, 1.0, 0.0).astype(jnp.bfloat16)
        nbr = jnp.dot(onehot, xb_all, preferred_element_type=jnp.float32)
        w = jnp.dot(nbr.astype(jnp.bfloat16), wd,
                    preferred_element_type=jnp.float32)
        wmax = jnp.maximum(wmax, w)
        wmin = jnp.minimum(wmin, w)

    s = s_ref[...]
    pick = jnp.where(s > 0, wmax, wmin)
    y = _lrelu((base + pick) * s + b_ref[...])
    out_ref[...] = y.astype(out_ref.dtype)


def _edge_conv(x, k, wc, wd, scale, bias, tn):
    B, N, C = x.shape
    Cout = wc.shape[1]
    body = functools.partial(_edge_conv_body, k=k)
    return pl.pallas_call(
        body,
        out_shape=jax.ShapeDtypeStruct((B, N, Cout), jnp.bfloat16),
        grid=(B, N // tn),
        in_specs=[
            pl.BlockSpec((None, N, C), lambda b, i: (b, 0, 0)),
            pl.BlockSpec((C, Cout), lambda b, i: (0, 0)),
            pl.BlockSpec((C, Cout), lambda b, i: (0, 0)),
            pl.BlockSpec((1, Cout), lambda b, i: (0, 0)),
            pl.BlockSpec((1, Cout), lambda b, i: (0, 0)),
        ],
        out_specs=pl.BlockSpec((None, tn, Cout), lambda b, i: (b, i, 0)),
        compiler_params=_cparams(("parallel", "parallel")),
    )(x, wc, wd, scale, bias)


def _global_feat_body(x1_ref, x2_ref, x3_ref, x4_ref,
                      w5a_ref, w5b_ref, w5c_ref, w5d_ref, s5_ref, b5_ref,
                      w1g_ref, g_ref, acc_ref):
    nt = pl.program_id(1)

    @pl.when(nt == 0)
    def _():
        acc_ref[...] = jnp.full(acc_ref.shape, -jnp.inf, acc_ref.dtype)

    y = (jnp.dot(x1_ref[...], w5a_ref[...], preferred_element_type=jnp.float32)
         + jnp.dot(x2_ref[...], w5b_ref[...], preferred_element_type=jnp.float32)
         + jnp.dot(x3_ref[...], w5c_ref[...], preferred_element_type=jnp.float32)
         + jnp.dot(x4_ref[...], w5d_ref[...], preferred_element_type=jnp.float32))
    y = _lrelu(y * s5_ref[...] + b5_ref[...])

    tn = y.shape[0]
    r = y[0:8]
    for gi in range(1, tn // 8):
        r = jnp.maximum(r, y[gi * 8:(gi + 1) * 8])
    acc_ref[...] = jnp.maximum(acc_ref[...], r)

    @pl.when(nt == pl.num_programs(1) - 1)
    def _():
        gmax = jnp.max(acc_ref[...], axis=0, keepdims=True)
        g_ref[...] = jnp.dot(gmax.astype(jnp.bfloat16), w1g_ref[...],
                             preferred_element_type=jnp.float32)


def _global_feature(xs, w5s, s5, b5, w1g, tn):
    B, N, _ = xs[0].shape
    Cmid = w5s[0].shape[1]
    Cg = w1g.shape[1]
    in_specs = [pl.BlockSpec((None, tn, x.shape[2]), lambda b, i: (b, i, 0))
                for x in xs]
    in_specs += [pl.BlockSpec(w.shape, lambda b, i: (0, 0)) for w in w5s]
    in_specs += [pl.BlockSpec((1, Cmid), lambda b, i: (0, 0))] * 2
    in_specs += [pl.BlockSpec((Cmid, Cg), lambda b, i: (0, 0))]
    return pl.pallas_call(
        _global_feat_body,
        out_shape=jax.ShapeDtypeStruct((B, 1, Cg), jnp.float32),
        grid=(B, N // tn),
        in_specs=in_specs,
        out_specs=pl.BlockSpec((None, 1, Cg), lambda b, i: (b, 0, 0)),
        scratch_shapes=[pltpu.VMEM((8, Cmid), jnp.float32)],
        compiler_params=_cparams(("parallel", "arbitrary")),
    )(*xs, *w5s, s5, b5, w1g)


def _seg_head_body(x1_ref, x2_ref, x3_ref, x4_ref, g_ref,
                   w1a_ref, w1b_ref, w1c_ref, w1d_ref, s1_ref, b1_ref,
                   w2_ref, s2_ref, b2_ref, w3_ref, out_ref):
    y = (jnp.dot(x1_ref[...], w1a_ref[...], preferred_element_type=jnp.float32)
         + jnp.dot(x2_ref[...], w1b_ref[...], preferred_element_type=jnp.float32)
         + jnp.dot(x3_ref[...], w1c_ref[...], preferred_element_type=jnp.float32)
         + jnp.dot(x4_ref[...], w1d_ref[...], preferred_element_type=jnp.float32))
    y = _lrelu((y + g_ref[...]) * s1_ref[...] + b1_ref[...])
    y = jnp.dot(y.astype(jnp.bfloat16), w2_ref[...],
                preferred_element_type=jnp.float32)
    y = _lrelu(y * s2_ref[...] + b2_ref[...])
    out_ref[...] = jnp.dot(y.astype(jnp.bfloat16), w3_ref[...],
                           preferred_element_type=jnp.float32)


def _seg_head(xs, g, w1s, s1, b1, w2, s2, b2, w3p, tn):
    B, N, _ = xs[0].shape
    C1 = w1s[0].shape[1]
    C2 = w2.shape[1]
    Cp = w3p.shape[1]
    in_specs = [pl.BlockSpec((None, tn, x.shape[2]), lambda b, i: (b, i, 0))
                for x in xs]
    in_specs += [pl.BlockSpec((None, 1, C1), lambda b, i: (b, 0, 0))]
    in_specs += [pl.BlockSpec(w.shape, lambda b, i: (0, 0)) for w in w1s]
    in_specs += [pl.BlockSpec((1, C1), lambda b, i: (0, 0))] * 2
    in_specs += [pl.BlockSpec(w2.shape, lambda b, i: (0, 0))]
    in_specs += [pl.BlockSpec((1, C2), lambda b, i: (0, 0))] * 2
    in_specs += [pl.BlockSpec(w3p.shape, lambda b, i: (0, 0))]
    return pl.pallas_call(
        _seg_head_body,
        out_shape=jax.ShapeDtypeStruct((B, N, Cp), jnp.float32),
        grid=(B, N // tn),
        in_specs=in_specs,
        out_specs=pl.BlockSpec((None, tn, Cp), lambda b, i: (b, i, 0)),
        compiler_params=_cparams(("parallel", "parallel")),
    )(*xs, g, *w1s, s1, b1, w2, s2, b2, w3p)


@jax.jit
def kernel(x, c1_wc, c1_wd, c1_s, c1_b, c2_wc, c2_wd, c2_s, c2_b,
           c3_wc, c3_wd, c3_s, c3_b, c4_wc, c4_wd, c4_s, c4_b,
           c5_a, c5_b2, c5_c, c5_d, c5_s, c5_bias, w1g,
           sg1_a, sg1_b, sg1_c, sg1_d, sg1_s, sg1_bias,
           sg2_w, sg2_s, sg2_bias, w3p):
    K = 20
    NUM_CLASSES = 13
    B, N, F = x.shape

    fpad = c1_wc.shape[0]
    if fpad != F:
        x = jnp.concatenate([x, jnp.zeros((B, N, fpad - F), x.dtype)], axis=-1)

    tn = 256
    x1 = _edge_conv(x, K, c1_wc, c1_wd, c1_s, c1_b, tn)
    x2 = _edge_conv(x1, K, c2_wc, c2_wd, c2_s, c2_b, tn)
    x3 = _edge_conv(x2, K, c3_wc, c3_wd, c3_s, c3_b, tn)
    x4 = _edge_conv(x3, K, c4_wc, c4_wd, c4_s, c4_b, tn)
    xs = (x1, x2, x3, x4)

    ht = 512
    g = _global_feature(xs, (c5_a, c5_b2, c5_c, c5_d), c5_s, c5_bias, w1g, ht)
    logits = _seg_head(xs, g, (sg1_a, sg1_b, sg1_c, sg1_d), sg1_s, sg1_bias,
                       sg2_w, sg2_s, sg2_bias, w3p, ht)
    return logits[:, :, :NUM_CLASSES]


# speedup vs baseline: 1.9047x; 1.7093x over previous
import functools

import jax
import jax.numpy as jnp
from jax import lax
from jax.experimental import pallas as pl
from jax.experimental.pallas import tpu as pltpu

LEAKY_SLOPE = 0.2

_VMEM_LIMIT_BYTES = int(min((64 * 1024 * 1024) * 3 // 4, 100 * 1024 * 1024))


def _cparams(semantics):
    return pltpu.CompilerParams(dimension_semantics=semantics,
                                vmem_limit_bytes=_VMEM_LIMIT_BYTES)


def _lrelu(y):
    return jnp.where(y > 0, y, LEAKY_SLOPE * y)


def _edge_conv_t_body(x_ref, xt_ref, wct_ref, wdt_ref, s_ref, b_ref,
                      out_ref, outt_ref, *, k):
    n = x_ref.shape[0]
    cout, tn = outt_ref.shape
    row0 = pl.multiple_of(pl.program_id(1) * tn, tn)

    x_all = x_ref[...]
    xf_all = x_all.astype(jnp.float32)
    xt_all = xt_ref[...]
    xtb_all = xt_all.astype(jnp.bfloat16)
    xt_tile = xt_ref[:, pl.ds(row0, tn)]
    xtt_f = xt_tile.astype(jnp.float32)
    xtt_b = xt_tile.astype(jnp.bfloat16)

    inner = lax.dot_general(xf_all, xtt_f, (((1,), (0,)), ((), ())),
                            preferred_element_type=jnp.float32)
    sq_t = jnp.sum(xtt_f * xtt_f, axis=0, keepdims=True)
    sq_a = jnp.sum(xf_all * xf_all, axis=1, keepdims=True)
    d = 2.0 * inner - sq_t - sq_a

    row = lax.broadcasted_iota(jnp.int32, (n, tn), 0)

    wdt = wdt_ref[...]
    self_row = row0 + lax.broadcasted_iota(jnp.int32, (1, tn), 1)
    d = jnp.where(row == self_row, -jnp.inf, d)

    baset = (jnp.dot(wct_ref[...], xtt_b, preferred_element_type=jnp.float32)
             - jnp.dot(wdt, xtt_b, preferred_element_type=jnp.float32))

    w_self = jnp.dot(wdt, xtt_b, preferred_element_type=jnp.float32)
    wmax = w_self
    wmin = w_self

    for _ in range(k - 1):
        idx = jnp.argmax(d, axis=0, keepdims=True)
        hit = row == idx
        d = jnp.where(hit, -jnp.inf, d)
        onehot = jnp.where(hit, 1.0, 0.0).astype(jnp.bfloat16)
        nbrt = lax.dot_general(xtb_all, onehot, (((1,), (0,)), ((), ())),
                               preferred_element_type=jnp.float32)
        w = jnp.dot(wdt, nbrt.astype(jnp.bfloat16),
                    preferred_element_type=jnp.float32)
        wmax = jnp.maximum(wmax, w)
        wmin = jnp.minimum(wmin, w)

    st = s_ref[...].T
    bt = b_ref[...].T
    pick = jnp.where(st > 0, wmax, wmin)
    yt = _lrelu((baset + pick) * st + bt)
    outt_ref[...] = yt.astype(outt_ref.dtype)
    out_ref[...] = yt.T.astype(out_ref.dtype)


def _edge_conv_t(x, xt, k, wct, wdt, scale, bias, tn):
    B, N, C = x.shape
    Cout = wct.shape[0]
    body = functools.partial(_edge_conv_t_body, k=k)
    return pl.pallas_call(
        body,
        out_shape=(jax.ShapeDtypeStruct((B, N, Cout), jnp.bfloat16),
                   jax.ShapeDtypeStruct((B, Cout, N), jnp.bfloat16)),
        grid=(B, N // tn),
        in_specs=[
            pl.BlockSpec((None, N, C), lambda b, i: (b, 0, 0)),
            pl.BlockSpec((None, C, N), lambda b, i: (b, 0, 0)),
            pl.BlockSpec((Cout, C), lambda b, i: (0, 0)),
            pl.BlockSpec((Cout, C), lambda b, i: (0, 0)),
            pl.BlockSpec((1, Cout), lambda b, i: (0, 0)),
            pl.BlockSpec((1, Cout), lambda b, i: (0, 0)),
        ],
        out_specs=(pl.BlockSpec((None, tn, Cout), lambda b, i: (b, i, 0)),
                   pl.BlockSpec((None, Cout, tn), lambda b, i: (b, 0, i))),
        compiler_params=_cparams(("parallel", "parallel")),
    )(x, xt, wct, wdt, scale, bias)


def _global_feat_body(x1_ref, x2_ref, x3_ref, x4_ref,
                      w5a_ref, w5b_ref, w5c_ref, w5d_ref, s5_ref, b5_ref,
                      w1g_ref, g_ref, acc_ref):
    nt = pl.program_id(1)

    @pl.when(nt == 0)
    def _():
        acc_ref[...] = jnp.full(acc_ref.shape, -jnp.inf, acc_ref.dtype)

    y = (jnp.dot(x1_ref[...], w5a_ref[...], preferred_element_type=jnp.float32)
         + jnp.dot(x2_ref[...], w5b_ref[...], preferred_element_type=jnp.float32)
         + jnp.dot(x3_ref[...], w5c_ref[...], preferred_element_type=jnp.float32)
         + jnp.dot(x4_ref[...], w5d_ref[...], preferred_element_type=jnp.float32))
    y = _lrelu(y * s5_ref[...] + b5_ref[...])

    tn = y.shape[0]
    r = y[0:8]
    for gi in range(1, tn // 8):
        r = jnp.maximum(r, y[gi * 8:(gi + 1) * 8])
    acc_ref[...] = jnp.maximum(acc_ref[...], r)

    @pl.when(nt == pl.num_programs(1) - 1)
    def _():
        gmax = jnp.max(acc_ref[...], axis=0, keepdims=True)
        g_ref[...] = jnp.dot(gmax.astype(jnp.bfloat16), w1g_ref[...],
                             preferred_element_type=jnp.float32)


def _global_feature(xs, w5s, s5, b5, w1g, tn):
    B, N, _ = xs[0].shape
    Cmid = w5s[0].shape[1]
    Cg = w1g.shape[1]
    in_specs = [pl.BlockSpec((None, tn, x.shape[2]), lambda b, i: (b, i, 0))
                for x in xs]
    in_specs += [pl.BlockSpec(w.shape, lambda b, i: (0, 0)) for w in w5s]
    in_specs += [pl.BlockSpec((1, Cmid), lambda b, i: (0, 0))] * 2
    in_specs += [pl.BlockSpec((Cmid, Cg), lambda b, i: (0, 0))]
    return pl.pallas_call(
        _global_feat_body,
        out_shape=jax.ShapeDtypeStruct((B, 1, Cg), jnp.float32),
        grid=(B, N // tn),
        in_specs=in_specs,
        out_specs=pl.BlockSpec((None, 1, Cg), lambda b, i: (b, 0, 0)),
        scratch_shapes=[pltpu.VMEM((8, Cmid), jnp.float32)],
        compiler_params=_cparams(("parallel", "arbitrary")),
    )(*xs, *w5s, s5, b5, w1g)


def _seg_head_body(x1_ref, x2_ref, x3_ref, x4_ref, g_ref,
                   w1a_ref, w1b_ref, w1c_ref, w1d_ref, s1_ref, b1_ref,
                   w2_ref, s2_ref, b2_ref, w3_ref, out_ref):
    y = (jnp.dot(x1_ref[...], w1a_ref[...], preferred_element_type=jnp.float32)
         + jnp.dot(x2_ref[...], w1b_ref[...], preferred_element_type=jnp.float32)
         + jnp.dot(x3_ref[...], w1c_ref[...], preferred_element_type=jnp.float32)
         + jnp.dot(x4_ref[...], w1d_ref[...], preferred_element_type=jnp.float32))
    y = _lrelu((y + g_ref[...]) * s1_ref[...] + b1_ref[...])
    y = jnp.dot(y.astype(jnp.bfloat16), w2_ref[...],
                preferred_element_type=jnp.float32)
    y = _lrelu(y * s2_ref[...] + b2_ref[...])
    out_ref[...] = jnp.dot(y.astype(jnp.bfloat16), w3_ref[...],
                           preferred_element_type=jnp.float32)


def _seg_head(xs, g, w1s, s1, b1, w2, s2, b2, w3p, tn):
    B, N, _ = xs[0].shape
    C1 = w1s[0].shape[1]
    C2 = w2.shape[1]
    Cp = w3p.shape[1]
    in_specs = [pl.BlockSpec((None, tn, x.shape[2]), lambda b, i: (b, i, 0))
                for x in xs]
    in_specs += [pl.BlockSpec((None, 1, C1), lambda b, i: (b, 0, 0))]
    in_specs += [pl.BlockSpec(w.shape, lambda b, i: (0, 0)) for w in w1s]
    in_specs += [pl.BlockSpec((1, C1), lambda b, i: (0, 0))] * 2
    in_specs += [pl.BlockSpec(w2.shape, lambda b, i: (0, 0))]
    in_specs += [pl.BlockSpec((1, C2), lambda b, i: (0, 0))] * 2
    in_specs += [pl.BlockSpec(w3p.shape, lambda b, i: (0, 0))]
    return pl.pallas_call(
        _seg_head_body,
        out_shape=jax.ShapeDtypeStruct((B, N, Cp), jnp.float32),
        grid=(B, N // tn),
        in_specs=in_specs,
        out_specs=pl.BlockSpec((None, tn, Cp), lambda b, i: (b, i, 0)),
        compiler_params=_cparams(("parallel", "parallel")),
    )(*xs, g, *w1s, s1, b1, w2, s2, b2, w3p)


@jax.jit
def kernel(x, c1_wc, c1_wd, c1_s, c1_b, c2_wc, c2_wd, c2_s, c2_b,
           c3_wc, c3_wd, c3_s, c3_b, c4_wc, c4_wd, c4_s, c4_b,
           c5_a, c5_b2, c5_c, c5_d, c5_s, c5_bias, w1g,
           sg1_a, sg1_b, sg1_c, sg1_d, sg1_s, sg1_bias,
           sg2_w, sg2_s, sg2_bias, w3p):
    K = 20
    NUM_CLASSES = 13
    B, N, F = x.shape

    fpad = c1_wc.shape[0]
    if fpad != F:
        x = jnp.concatenate([x, jnp.zeros((B, N, fpad - F), x.dtype)], axis=-1)
    xt = jnp.swapaxes(x, 1, 2)

    tn = 256
    x1, x1t = _edge_conv_t(x, xt, K, c1_wc.T, c1_wd.T, c1_s, c1_b, tn)
    x2, x2t = _edge_conv_t(x1, x1t, K, c2_wc.T, c2_wd.T, c2_s, c2_b, tn)
    x3, x3t = _edge_conv_t(x2, x2t, K, c3_wc.T, c3_wd.T, c3_s, c3_b, tn)
    x4, _ = _edge_conv_t(x3, x3t, K, c4_wc.T, c4_wd.T, c4_s, c4_b, tn)
    xs = (x1, x2, x3, x4)

    ht = 512
    g = _global_feature(xs, (c5_a, c5_b2, c5_c, c5_d), c5_s, c5_bias, w1g, ht)
    logits = _seg_head(xs, g, (sg1_a, sg1_b, sg1_c, sg1_d), sg1_s, sg1_bias,
                       sg2_w, sg2_s, sg2_bias, w3p, ht)
    return logits[:, :, :NUM_CLASSES]


# speedup vs baseline: 1.9305x; 1.0135x over previous
import functools

import jax
import jax.numpy as jnp
from jax import lax
from jax.experimental import pallas as pl
from jax.experimental.pallas import tpu as pltpu

LEAKY_SLOPE = 0.2

_VMEM_LIMIT_BYTES = int(min((64 * 1024 * 1024) * 3 // 4, 100 * 1024 * 1024))


def _cparams(semantics):
    return pltpu.CompilerParams(dimension_semantics=semantics,
                                vmem_limit_bytes=_VMEM_LIMIT_BYTES)


def _lrelu(y):
    return jnp.where(y > 0, y, LEAKY_SLOPE * y)


def _edge_conv_t_body(x_ref, xt_ref, wct_ref, wdt_ref, s_ref, b_ref,
                      out_ref, outt_ref, *, k):
    n = x_ref.shape[0]
    cout, tn = outt_ref.shape
    row0 = pl.multiple_of(pl.program_id(1) * tn, tn)

    x_all = x_ref[...]
    xf_all = x_all.astype(jnp.float32)
    xt_all = xt_ref[...]
    xtb_all = xt_all.astype(jnp.bfloat16)
    xt_tile = xt_ref[:, pl.ds(row0, tn)]
    xtt_f = xt_tile.astype(jnp.float32)
    xtt_b = xt_tile.astype(jnp.bfloat16)

    inner = lax.dot_general(xf_all, xtt_f, (((1,), (0,)), ((), ())),
                            preferred_element_type=jnp.float32)
    sq_t = jnp.sum(xtt_f * xtt_f, axis=0, keepdims=True)
    sq_a = jnp.sum(xf_all * xf_all, axis=1, keepdims=True)
    d = 2.0 * inner - sq_t - sq_a

    row = lax.broadcasted_iota(jnp.int32, (n, tn), 0)

    wdt = wdt_ref[...]
    self_row = row0 + lax.broadcasted_iota(jnp.int32, (1, tn), 1)
    d = jnp.where(row == self_row, -jnp.inf, d)

    baset = (jnp.dot(wct_ref[...], xtt_b, preferred_element_type=jnp.float32)
             - jnp.dot(wdt, xtt_b, preferred_element_type=jnp.float32))

    w_self = jnp.dot(wdt, xtt_b, preferred_element_type=jnp.float32)
    wmax = w_self
    wmin = w_self

    for _ in range(k - 1):
        idx = jnp.argmax(d, axis=0, keepdims=True)
        hit = row == idx
        d = jnp.where(hit, -jnp.inf, d)
        onehot = jnp.where(hit, 1.0, 0.0).astype(jnp.bfloat16)
        nbrt = lax.dot_general(xtb_all, onehot, (((1,), (0,)), ((), ())),
                               preferred_element_type=jnp.float32)
        w = jnp.dot(wdt, nbrt.astype(jnp.bfloat16),
                    preferred_element_type=jnp.float32)
        wmax = jnp.maximum(wmax, w)
        wmin = jnp.minimum(wmin, w)

    st = s_ref[...].T
    bt = b_ref[...].T
    pick = jnp.where(st > 0, wmax, wmin)
    yt = _lrelu((baset + pick) * st + bt)
    outt_ref[...] = yt.astype(outt_ref.dtype)
    out_ref[...] = yt.T.astype(out_ref.dtype)


def _edge_conv_t(x, xt, k, wct, wdt, scale, bias, tn):
    B, N, C = x.shape
    Cout = wct.shape[0]
    body = functools.partial(_edge_conv_t_body, k=k)
    return pl.pallas_call(
        body,
        out_shape=(jax.ShapeDtypeStruct((B, N, Cout), jnp.bfloat16),
                   jax.ShapeDtypeStruct((B, Cout, N), jnp.bfloat16)),
        grid=(B, N // tn),
        in_specs=[
            pl.BlockSpec((None, N, C), lambda b, i: (b, 0, 0)),
            pl.BlockSpec((None, C, N), lambda b, i: (b, 0, 0)),
            pl.BlockSpec((Cout, C), lambda b, i: (0, 0)),
            pl.BlockSpec((Cout, C), lambda b, i: (0, 0)),
            pl.BlockSpec((1, Cout), lambda b, i: (0, 0)),
            pl.BlockSpec((1, Cout), lambda b, i: (0, 0)),
        ],
        out_specs=(pl.BlockSpec((None, tn, Cout), lambda b, i: (b, i, 0)),
                   pl.BlockSpec((None, Cout, tn), lambda b, i: (b, 0, i))),
        compiler_params=_cparams(("parallel", "parallel")),
    )(x, xt, wct, wdt, scale, bias)


def _global_feat_body(x1_ref, x2_ref, x3_ref, x4_ref,
                      w5a_ref, w5b_ref, w5c_ref, w5d_ref, s5_ref, b5_ref,
                      w1g_ref, g_ref, acc_ref):
    nt = pl.program_id(1)

    @pl.when(nt == 0)
    def _():
        acc_ref[...] = jnp.full(acc_ref.shape, -jnp.inf, acc_ref.dtype)

    y = (jnp.dot(x1_ref[...], w5a_ref[...], preferred_element_type=jnp.float32)
         + jnp.dot(x2_ref[...], w5b_ref[...], preferred_element_type=jnp.float32)
         + jnp.dot(x3_ref[...], w5c_ref[...], preferred_element_type=jnp.float32)
         + jnp.dot(x4_ref[...], w5d_ref[...], preferred_element_type=jnp.float32))
    y = _lrelu(y * s5_ref[...] + b5_ref[...])

    tn = y.shape[0]
    r = y[0:8]
    for gi in range(1, tn // 8):
        r = jnp.maximum(r, y[gi * 8:(gi + 1) * 8])
    acc_ref[...] = jnp.maximum(acc_ref[...], r)

    @pl.when(nt == pl.num_programs(1) - 1)
    def _():
        gmax = jnp.max(acc_ref[...], axis=0, keepdims=True)
        g_ref[...] = jnp.dot(gmax.astype(jnp.bfloat16), w1g_ref[...],
                             preferred_element_type=jnp.float32)


def _global_feature(xs, w5s, s5, b5, w1g, tn):
    B, N, _ = xs[0].shape
    Cmid = w5s[0].shape[1]
    Cg = w1g.shape[1]
    in_specs = [pl.BlockSpec((None, tn, x.shape[2]), lambda b, i: (b, i, 0))
                for x in xs]
    in_specs += [pl.BlockSpec(w.shape, lambda b, i: (0, 0)) for w in w5s]
    in_specs += [pl.BlockSpec((1, Cmid), lambda b, i: (0, 0))] * 2
    in_specs += [pl.BlockSpec((Cmid, Cg), lambda b, i: (0, 0))]
    return pl.pallas_call(
        _global_feat_body,
        out_shape=jax.ShapeDtypeStruct((B, 1, Cg), jnp.float32),
        grid=(B, N // tn),
        in_specs=in_specs,
        out_specs=pl.BlockSpec((None, 1, Cg), lambda b, i: (b, 0, 0)),
        scratch_shapes=[pltpu.VMEM((8, Cmid), jnp.float32)],
        compiler_params=_cparams(("parallel", "arbitrary")),
    )(*xs, *w5s, s5, b5, w1g)


def _seg_head_body(x1_ref, x2_ref, x3_ref, x4_ref, g_ref,
                   w1a_ref, w1b_ref, w1c_ref, w1d_ref, s1_ref, b1_ref,
                   w2_ref, s2_ref, b2_ref, w3_ref, out_ref):
    y = (jnp.dot(x1_ref[...], w1a_ref[...], preferred_element_type=jnp.float32)
         + jnp.dot(x2_ref[...], w1b_ref[...], preferred_element_type=jnp.float32)
         + jnp.dot(x3_ref[...], w1c_ref[...], preferred_element_type=jnp.float32)
         + jnp.dot(x4_ref[...], w1d_ref[...], preferred_element_type=jnp.float32))
    y = _lrelu((y + g_ref[...]) * s1_ref[...] + b1_ref[...])
    y = jnp.dot(y.astype(jnp.bfloat16), w2_ref[...],
                preferred_element_type=jnp.float32)
    y = _lrelu(y * s2_ref[...] + b2_ref[...])
    out_ref[...] = jnp.dot(y.astype(jnp.bfloat16), w3_ref[...],
                           preferred_element_type=jnp.float32)


def _seg_head(xs, g, w1s, s1, b1, w2, s2, b2, w3p, tn):
    B, N, _ = xs[0].shape
    C1 = w1s[0].shape[1]
    C2 = w2.shape[1]
    Cp = w3p.shape[1]
    in_specs = [pl.BlockSpec((None, tn, x.shape[2]), lambda b, i: (b, i, 0))
                for x in xs]
    in_specs += [pl.BlockSpec((None, 1, C1), lambda b, i: (b, 0, 0))]
    in_specs += [pl.BlockSpec(w.shape, lambda b, i: (0, 0)) for w in w1s]
    in_specs += [pl.BlockSpec((1, C1), lambda b, i: (0, 0))] * 2
    in_specs += [pl.BlockSpec(w2.shape, lambda b, i: (0, 0))]
    in_specs += [pl.BlockSpec((1, C2), lambda b, i: (0, 0))] * 2
    in_specs += [pl.BlockSpec(w3p.shape, lambda b, i: (0, 0))]
    return pl.pallas_call(
        _seg_head_body,
        out_shape=jax.ShapeDtypeStruct((B, N, Cp), jnp.float32),
        grid=(B, N // tn),
        in_specs=in_specs,
        out_specs=pl.BlockSpec((None, tn, Cp), lambda b, i: (b, i, 0)),
        compiler_params=_cparams(("parallel", "parallel")),
    )(*xs, g, *w1s, s1, b1, w2, s2, b2, w3p)


@jax.jit
def kernel(x, c1_wc, c1_wd, c1_s, c1_b, c2_wc, c2_wd, c2_s, c2_b,
           c3_wc, c3_wd, c3_s, c3_b, c4_wc, c4_wd, c4_s, c4_b,
           c5_a, c5_b2, c5_c, c5_d, c5_s, c5_bias, w1g,
           sg1_a, sg1_b, sg1_c, sg1_d, sg1_s, sg1_bias,
           sg2_w, sg2_s, sg2_bias, w3p):
    K = 20
    NUM_CLASSES = 13
    B, N, F = x.shape

    fpad = c1_wc.shape[0]
    if fpad != F:
        x = jnp.concatenate([x, jnp.zeros((B, N, fpad - F), x.dtype)], axis=-1)
    xt = jnp.swapaxes(x, 1, 2)

    tn = 512
    x1, x1t = _edge_conv_t(x, xt, K, c1_wc.T, c1_wd.T, c1_s, c1_b, tn)
    x2, x2t = _edge_conv_t(x1, x1t, K, c2_wc.T, c2_wd.T, c2_s, c2_b, tn)
    x3, x3t = _edge_conv_t(x2, x2t, K, c3_wc.T, c3_wd.T, c3_s, c3_b, tn)
    x4, _ = _edge_conv_t(x3, x3t, K, c4_wc.T, c4_wd.T, c4_s, c4_b, tn)
    xs = (x1, x2, x3, x4)

    ht = 512
    g = _global_feature(xs, (c5_a, c5_b2, c5_c, c5_d), c5_s, c5_bias, w1g, ht)
    logits = _seg_head(xs, g, (sg1_a, sg1_b, sg1_c, sg1_d), sg1_s, sg1_bias,
                       sg2_w, sg2_s, sg2_bias, w3p, ht)
    return logits[:, :, :NUM_CLASSES]


# speedup vs baseline: 1.9320x; 1.0008x over previous
import functools

import jax
import jax.numpy as jnp
from jax import lax
from jax.experimental import pallas as pl
from jax.experimental.pallas import tpu as pltpu

LEAKY_SLOPE = 0.2

_VMEM_LIMIT_BYTES = int(min((64 * 1024 * 1024) * 3 // 4, 100 * 1024 * 1024))


def _cparams(semantics):
    return pltpu.CompilerParams(dimension_semantics=semantics,
                                vmem_limit_bytes=_VMEM_LIMIT_BYTES)


def _lrelu(y):
    return jnp.where(y > 0, y, LEAKY_SLOPE * y)


def _edge_conv_t_body(x_ref, xt_ref, wct_ref, wdt_ref, s_ref, b_ref,
                      out_ref, *rest, k, want_xt):
    outt_ref = rest[0] if want_xt else None
    n = x_ref.shape[0]
    tn, cout = out_ref.shape
    row0 = pl.multiple_of(pl.program_id(1) * tn, tn)

    x_all = x_ref[...]
    xf_all = x_all.astype(jnp.float32)
    xtb_all = xt_ref[...].astype(jnp.bfloat16)
    xt_tile = xt_ref[:, pl.ds(row0, tn)]
    xtt_f = xt_tile.astype(jnp.float32)
    xtt_b = xt_tile.astype(jnp.bfloat16)

    inner = lax.dot_general(xf_all, xtt_f, (((1,), (0,)), ((), ())),
                            preferred_element_type=jnp.float32)
    sq_t = jnp.sum(xtt_f * xtt_f, axis=0, keepdims=True)
    sq_a = jnp.sum(xf_all * xf_all, axis=1, keepdims=True)
    d = 2.0 * inner - sq_t - sq_a

    row = lax.broadcasted_iota(jnp.int32, (n, tn), 0)

    wdt = wdt_ref[...]
    baset = (jnp.dot(wct_ref[...], xtt_b, preferred_element_type=jnp.float32)
             - jnp.dot(wdt, xtt_b, preferred_element_type=jnp.float32))
    w_self = jnp.dot(wdt, xtt_b, preferred_element_type=jnp.float32)
    wmax = w_self
    wmin = w_self

    idx = row0 + lax.broadcasted_iota(jnp.int32, (1, tn), 1)
    for t in range(k):
        hit = row == idx
        if t < k - 1:
            d = jnp.where(hit, -jnp.inf, d)
            nxt = jnp.argmax(d, axis=0, keepdims=True)
        if t > 0:
            onehot = jnp.where(hit, 1.0, 0.0).astype(jnp.bfloat16)
            nbrt = lax.dot_general(xtb_all, onehot, (((1,), (0,)), ((), ())),
                                   preferred_element_type=jnp.float32)
            w = jnp.dot(wdt, nbrt.astype(jnp.bfloat16),
                        preferred_element_type=jnp.float32)
            wmax = jnp.maximum(wmax, w)
            wmin = jnp.minimum(wmin, w)
        if t < k - 1:
            idx = nxt

    st = s_ref[...].T
    bt = b_ref[...].T
    pick = jnp.where(st > 0, wmax, wmin)
    yt = _lrelu((baset + pick) * st + bt)
    if want_xt:
        outt_ref[...] = yt.astype(outt_ref.dtype)
    out_ref[...] = yt.T.astype(out_ref.dtype)


def _edge_conv_t(x, xt, k, wct, wdt, scale, bias, tn, want_xt=True):
    B, N, C = x.shape
    Cout = wct.shape[0]
    body = functools.partial(_edge_conv_t_body, k=k, want_xt=want_xt)
    out_shape = [jax.ShapeDtypeStruct((B, N, Cout), jnp.bfloat16)]
    out_specs = [pl.BlockSpec((None, tn, Cout), lambda b, i: (b, i, 0))]
    if want_xt:
        out_shape.append(jax.ShapeDtypeStruct((B, Cout, N), jnp.bfloat16))
        out_specs.append(pl.BlockSpec((None, Cout, tn), lambda b, i: (b, 0, i)))
    res = pl.pallas_call(
        body,
        out_shape=tuple(out_shape),
        grid=(B, N // tn),
        in_specs=[
            pl.BlockSpec((None, N, C), lambda b, i: (b, 0, 0)),
            pl.BlockSpec((None, C, N), lambda b, i: (b, 0, 0)),
            pl.BlockSpec((Cout, C), lambda b, i: (0, 0)),
            pl.BlockSpec((Cout, C), lambda b, i: (0, 0)),
            pl.BlockSpec((1, Cout), lambda b, i: (0, 0)),
            pl.BlockSpec((1, Cout), lambda b, i: (0, 0)),
        ],
        out_specs=tuple(out_specs),
        compiler_params=_cparams(("parallel", "parallel")),
    )(x, xt, wct, wdt, scale, bias)
    if want_xt:
        return res
    return res[0], None


def _global_feat_body(x1_ref, x2_ref, x3_ref, x4_ref,
                      w5a_ref, w5b_ref, w5c_ref, w5d_ref, s5_ref, b5_ref,
                      w1g_ref, g_ref, acc_ref):
    nt = pl.program_id(1)

    @pl.when(nt == 0)
    def _():
        acc_ref[...] = jnp.full(acc_ref.shape, -jnp.inf, acc_ref.dtype)

    y = (jnp.dot(x1_ref[...], w5a_ref[...], preferred_element_type=jnp.float32)
         + jnp.dot(x2_ref[...], w5b_ref[...], preferred_element_type=jnp.float32)
         + jnp.dot(x3_ref[...], w5c_ref[...], preferred_element_type=jnp.float32)
         + jnp.dot(x4_ref[...], w5d_ref[...], preferred_element_type=jnp.float32))
    y = _lrelu(y * s5_ref[...] + b5_ref[...])

    tn = y.shape[0]
    r = y[0:8]
    for gi in range(1, tn // 8):
        r = jnp.maximum(r, y[gi * 8:(gi + 1) * 8])
    acc_ref[...] = jnp.maximum(acc_ref[...], r)

    @pl.when(nt == pl.num_programs(1) - 1)
    def _():
        gmax = jnp.max(acc_ref[...], axis=0, keepdims=True)
        g_ref[...] = jnp.dot(gmax.astype(jnp.bfloat16), w1g_ref[...],
                             preferred_element_type=jnp.float32)


def _global_feature(xs, w5s, s5, b5, w1g, tn):
    B, N, _ = xs[0].shape
    Cmid = w5s[0].shape[1]
    Cg = w1g.shape[1]
    in_specs = [pl.BlockSpec((None, tn, x.shape[2]), lambda b, i: (b, i, 0))
                for x in xs]
    in_specs += [pl.BlockSpec(w.shape, lambda b, i: (0, 0)) for w in w5s]
    in_specs += [pl.BlockSpec((1, Cmid), lambda b, i: (0, 0))] * 2
    in_specs += [pl.BlockSpec((Cmid, Cg), lambda b, i: (0, 0))]
    return pl.pallas_call(
        _global_feat_body,
        out_shape=jax.ShapeDtypeStruct((B, 1, Cg), jnp.float32),
        grid=(B, N // tn),
        in_specs=in_specs,
        out_specs=pl.BlockSpec((None, 1, Cg), lambda b, i: (b, 0, 0)),
        scratch_shapes=[pltpu.VMEM((8, Cmid), jnp.float32)],
        compiler_params=_cparams(("parallel", "arbitrary")),
    )(*xs, *w5s, s5, b5, w1g)


def _seg_head_body(x1_ref, x2_ref, x3_ref, x4_ref, g_ref,
                   w1a_ref, w1b_ref, w1c_ref, w1d_ref, s1_ref, b1_ref,
                   w2_ref, s2_ref, b2_ref, w3_ref, out_ref):
    y = (jnp.dot(x1_ref[...], w1a_ref[...], preferred_element_type=jnp.float32)
         + jnp.dot(x2_ref[...], w1b_ref[...], preferred_element_type=jnp.float32)
         + jnp.dot(x3_ref[...], w1c_ref[...], preferred_element_type=jnp.float32)
         + jnp.dot(x4_ref[...], w1d_ref[...], preferred_element_type=jnp.float32))
    y = _lrelu((y + g_ref[...]) * s1_ref[...] + b1_ref[...])
    y = jnp.dot(y.astype(jnp.bfloat16), w2_ref[...],
                preferred_element_type=jnp.float32)
    y = _lrelu(y * s2_ref[...] + b2_ref[...])
    out_ref[...] = jnp.dot(y.astype(jnp.bfloat16), w3_ref[...],
                           preferred_element_type=jnp.float32)


def _seg_head(xs, g, w1s, s1, b1, w2, s2, b2, w3p, tn):
    B, N, _ = xs[0].shape
    C1 = w1s[0].shape[1]
    C2 = w2.shape[1]
    Cp = w3p.shape[1]
    in_specs = [pl.BlockSpec((None, tn, x.shape[2]), lambda b, i: (b, i, 0))
                for x in xs]
    in_specs += [pl.BlockSpec((None, 1, C1), lambda b, i: (b, 0, 0))]
    in_specs += [pl.BlockSpec(w.shape, lambda b, i: (0, 0)) for w in w1s]
    in_specs += [pl.BlockSpec((1, C1), lambda b, i: (0, 0))] * 2
    in_specs += [pl.BlockSpec(w2.shape, lambda b, i: (0, 0))]
    in_specs += [pl.BlockSpec((1, C2), lambda b, i: (0, 0))] * 2
    in_specs += [pl.BlockSpec(w3p.shape, lambda b, i: (0, 0))]
    return pl.pallas_call(
        _seg_head_body,
        out_shape=jax.ShapeDtypeStruct((B, N, Cp), jnp.float32),
        grid=(B, N // tn),
        in_specs=in_specs,
        out_specs=pl.BlockSpec((None, tn, Cp), lambda b, i: (b, i, 0)),
        compiler_params=_cparams(("parallel", "parallel")),
    )(*xs, g, *w1s, s1, b1, w2, s2, b2, w3p)


@jax.jit
def kernel(x, c1_wc, c1_wd, c1_s, c1_b, c2_wc, c2_wd, c2_s, c2_b,
           c3_wc, c3_wd, c3_s, c3_b, c4_wc, c4_wd, c4_s, c4_b,
           c5_a, c5_b2, c5_c, c5_d, c5_s, c5_bias, w1g,
           sg1_a, sg1_b, sg1_c, sg1_d, sg1_s, sg1_bias,
           sg2_w, sg2_s, sg2_bias, w3p):
    K = 20
    NUM_CLASSES = 13
    B, N, F = x.shape

    fpad = c1_wc.shape[0]
    if fpad != F:
        x = jnp.concatenate([x, jnp.zeros((B, N, fpad - F), x.dtype)], axis=-1)
    xt = jnp.swapaxes(x, 1, 2)

    tn = 512
    x1, x1t = _edge_conv_t(x, xt, K, c1_wc.T, c1_wd.T, c1_s, c1_b, tn)
    x2, x2t = _edge_conv_t(x1, x1t, K, c2_wc.T, c2_wd.T, c2_s, c2_b, tn)
    x3, x3t = _edge_conv_t(x2, x2t, K, c3_wc.T, c3_wd.T, c3_s, c3_b, tn)
    x4, _ = _edge_conv_t(x3, x3t, K, c4_wc.T, c4_wd.T, c4_s, c4_b, tn,
                         want_xt=False)
    xs = (x1, x2, x3, x4)

    ht = 512
    g = _global_feature(xs, (c5_a, c5_b2, c5_c, c5_d), c5_s, c5_bias, w1g, ht)
    logits = _seg_head(xs, g, (sg1_a, sg1_b, sg1_c, sg1_d), sg1_s, sg1_bias,
                       sg2_w, sg2_s, sg2_bias, w3p, ht)
    return logits[:, :, :NUM_CLASSES]
